# baseline (speedup 1.0000x reference)
import jax
import jax.numpy as jnp
from jax.experimental import pallas as pl
from jax.experimental.pallas import tpu as pltpu

_ROWS = 1000
_LANES = 640
_N_CHUNKS = 5
_CH = _ROWS // _N_CHUNKS


def _dma_pipe_kernel(src, dst, buf0, buf1, buf2, buf3, buf4, in_sems, out_sems):
    bufs = (buf0, buf1, buf2, buf3, buf4)

    def in_copy(i):
        return pltpu.make_async_copy(
            src.at[pl.ds(i * _CH, _CH)], bufs[i], in_sems.at[i])

    def out_copy(i):
        return pltpu.make_async_copy(
            bufs[i], dst.at[pl.ds(i * _CH, _CH)], out_sems.at[i])

    for i in range(_N_CHUNKS):
        in_copy(i).start()
    for i in range(_N_CHUNKS):
        in_copy(i).wait()
        out_copy(i).start()
    for i in range(_N_CHUNKS):
        out_copy(i).wait()


def kernel(node_features, edges, layer_lengths, W1, b1, W2, b2):
    flat = edges.reshape(_ROWS, _LANES)
    out = pl.pallas_call(
        _dma_pipe_kernel,
        in_specs=[pl.BlockSpec(memory_space=pl.ANY)],
        out_specs=pl.BlockSpec(memory_space=pl.ANY),
        out_shape=jax.ShapeDtypeStruct(flat.shape, flat.dtype),
        scratch_shapes=[
            pltpu.VMEM((_CH, _LANES), jnp.int32),
            pltpu.VMEM((_CH, _LANES), jnp.int32),
            pltpu.VMEM((_CH, _LANES), jnp.int32),
            pltpu.VMEM((_CH, _LANES), jnp.int32),
            pltpu.VMEM((_CH, _LANES), jnp.int32),
            pltpu.SemaphoreType.DMA((_N_CHUNKS,)),
            pltpu.SemaphoreType.DMA((_N_CHUNKS,)),
        ],
    )(flat)
    return out.reshape(edges.shape)


# speedup vs baseline: 1.0023x; 1.0023x over previous
import jax
import jax.numpy as jnp
from jax.experimental import pallas as pl
from jax.experimental.pallas import tpu as pltpu

_N = 640000
_N_CHUNKS = 5
_CH = _N // _N_CHUNKS


def _dma_pipe_kernel(src, dst, buf0, buf1, buf2, buf3, buf4, in_sems, out_sems):
    bufs = (buf0, buf1, buf2, buf3, buf4)

    def in_copy(i):
        return pltpu.make_async_copy(
            src.at[pl.ds(i * _CH, _CH)], bufs[i], in_sems.at[i])

    def out_copy(i):
        return pltpu.make_async_copy(
            bufs[i], dst.at[pl.ds(i * _CH, _CH)], out_sems.at[i])

    for i in range(_N_CHUNKS):
        in_copy(i).start()
    for i in range(_N_CHUNKS):
        in_copy(i).wait()
        out_copy(i).start()
    for i in range(_N_CHUNKS):
        out_copy(i).wait()


def kernel(node_features, edges, layer_lengths, W1, b1, W2, b2):
    flat = edges.reshape(_N)
    out = pl.pallas_call(
        _dma_pipe_kernel,
        in_specs=[pl.BlockSpec(memory_space=pl.ANY)],
        out_specs=pl.BlockSpec(memory_space=pl.ANY),
        out_shape=jax.ShapeDtypeStruct(flat.shape, flat.dtype),
        scratch_shapes=[
            pltpu.VMEM((_CH,), jnp.int32),
            pltpu.VMEM((_CH,), jnp.int32),
            pltpu.VMEM((_CH,), jnp.int32),
            pltpu.VMEM((_CH,), jnp.int32),
            pltpu.VMEM((_CH,), jnp.int32),
            pltpu.SemaphoreType.DMA((_N_CHUNKS,)),
            pltpu.SemaphoreType.DMA((_N_CHUNKS,)),
        ],
    )(flat)
    return out.reshape(edges.shape)
